# baseline (speedup 1.0000x reference)
import jax
import jax.numpy as jnp
from jax.experimental import pallas as pl


def kernel(inputs, pooling_argmax):
    b, h, w, c = inputs.shape
    out_h, out_w = h * 2, w * 2
    flat_out = out_h * out_w * c
    n = b * h * w * c
    v = inputs.reshape(n)
    a = pooling_argmax.reshape(n)
    a_s, v_s = jax.lax.sort((a, v), dimension=0, is_stable=False, num_keys=1)
    is_last = jnp.concatenate([a_s[1:] != a_s[:-1], jnp.array([True])])
    idx = jnp.where(is_last, a_s, b * flat_out)
    out = jnp.zeros((b * flat_out,), v.dtype).at[idx].set(v_s, mode="drop")
    return out.reshape(b, out_h, out_w, c)


# speedup vs baseline: 4.7327x; 4.7327x over previous
import functools

import jax
import jax.numpy as jnp
from jax import lax
from jax.experimental import pallas as pl
from jax.experimental.pallas import tpu as pltpu
from jax.experimental.pallas import tpu_sc as plsc

B, H, W, C = 8, 128, 128, 96
OUT_H, OUT_W = H * 2, W * 2
N = B * H * W * C
M = B * OUT_H * OUT_W * C
SEG = 65536
NSEG = M // SEG
NW = 32
SEGS_PER_TILE = NSEG // NW
K = 2048
NB = 784

_mesh = plsc.VectorSubcoreMesh(core_axis_name="c", subcore_axis_name="s")


@functools.partial(
    pl.kernel,
    out_type=jax.ShapeDtypeStruct((M,), jnp.float32),
    mesh=_mesh,
    scratch_types=[
        pltpu.VMEM((NB,), jnp.int32),
        pltpu.VMEM((SEG,), jnp.float32),
        pltpu.VMEM((K + 16,), jnp.int32),
        pltpu.VMEM((K + 16,), jnp.float32),
    ],
    compiler_params=pltpu.CompilerParams(needs_layout_passes=False),
)
def _scatter_sorted(keys_hbm, vals_hbm, bounds_hbm, zeros_hbm, out_hbm,
                    bnd_v, seg_v, kw_v, vw_v):
    wid = lax.axis_index("s") * 2 + lax.axis_index("c")
    pltpu.sync_copy(bounds_hbm, bnd_v)
    lanes = lax.iota(jnp.int32, 16)

    def bnd_at(i):
        vec = bnd_v[pl.ds((i >> 4) << 4, 16)]
        sel = jnp.where(lanes == (i & 15), vec, jnp.int32(-2147483647))
        return jnp.max(sel)

    def seg_body(sp, _):
        g = wid * SEGS_PER_TILE + sp
        seg_lo = pl.multiple_of(g * SEG, SEG)
        lo = bnd_at(g)
        hi = bnd_at(g + 1)
        lo_al = lo & jnp.int32(-8)
        trips = (hi - lo_al + (K - 1)) >> 11
        pltpu.sync_copy(zeros_hbm, seg_v)

        def win_body(w, _):
            off = pl.multiple_of(jnp.minimum(lo_al + w * K, N - K), 8)
            pltpu.sync_copy(keys_hbm.at[pl.ds(off, K)],
                            kw_v.at[pl.ds(0, K)])
            pltpu.sync_copy(vals_hbm.at[pl.ds(off, K)],
                            vw_v.at[pl.ds(0, K)])
            look = pl.multiple_of(jnp.minimum(off + K, N - 16), 8)
            pltpu.sync_copy(keys_hbm.at[pl.ds(look, 16)],
                            kw_v.at[pl.ds(K, 16)])
            tail = kw_v[pl.ds(K, 16)]
            is_end = (off + K) >= N
            kw_v[pl.ds(K, 16)] = jnp.where(is_end, jnp.int32(M), tail)

            def vec_body(j, _):
                cur = kw_v[pl.ds(j * 16, 16)]
                nxt = kw_v[pl.ds(j * 16 + 1, 16)]
                vals = vw_v[pl.ds(j * 16, 16)]
                is_last = cur != nxt
                in_seg = (cur >= seg_lo) & (cur < seg_lo + SEG)
                m = is_last & in_seg
                plsc.store_scatter(seg_v, [cur - seg_lo], vals, mask=m)
                return 0

            lax.fori_loop(0, K // 16, vec_body, 0)
            return 0

        lax.fori_loop(0, trips, win_body, 0)
        pltpu.sync_copy(seg_v, out_hbm.at[pl.ds(seg_lo, SEG)])
        return 0

    lax.fori_loop(0, SEGS_PER_TILE, seg_body, 0)


def kernel(inputs, pooling_argmax):
    v = inputs.reshape(N)
    a = pooling_argmax.reshape(N)
    a_s, v_s = lax.sort((a, v), dimension=0, is_stable=False, num_keys=1)
    seg_starts = jnp.arange(NSEG + 1, dtype=jnp.int32) * SEG
    bounds = jnp.searchsorted(a_s, seg_starts, side="left").astype(jnp.int32)
    bounds = jnp.concatenate(
        [bounds, jnp.full((NB - NSEG - 1,), N, jnp.int32)])
    assert bounds.shape == (NB,)
    zeros = jnp.zeros((SEG,), jnp.float32)
    out = _scatter_sorted(a_s, v_s, bounds, zeros)
    return out.reshape(B, OUT_H, OUT_W, C)


# speedup vs baseline: 4.8088x; 1.0161x over previous
import functools

import jax
import jax.numpy as jnp
from jax import lax
from jax.experimental import pallas as pl
from jax.experimental.pallas import tpu as pltpu
from jax.experimental.pallas import tpu_sc as plsc

B, H, W, C = 8, 128, 128, 96
OUT_H, OUT_W = H * 2, W * 2
N = B * H * W * C
M = B * OUT_H * OUT_W * C
SEG = 65536
NSEG = M // SEG
NW = 32
SEGS_PER_TILE = NSEG // NW
K = 8192
KSH = 13
NB = 784

_mesh = plsc.VectorSubcoreMesh(core_axis_name="c", subcore_axis_name="s")


@functools.partial(
    pl.kernel,
    out_type=jax.ShapeDtypeStruct((M,), jnp.float32),
    mesh=_mesh,
    scratch_types=[
        pltpu.VMEM((NB,), jnp.int32),
        pltpu.VMEM((SEG,), jnp.float32),
        pltpu.VMEM((K + 16,), jnp.int32),
        pltpu.VMEM((K + 16,), jnp.float32),
        pltpu.SemaphoreType.DMA,
        pltpu.SemaphoreType.DMA,
        pltpu.SemaphoreType.DMA,
    ],
    compiler_params=pltpu.CompilerParams(needs_layout_passes=False),
)
def _scatter_sorted(keys_hbm, vals_hbm, bounds_hbm, zeros_hbm, out_hbm,
                    bnd_v, seg_v, kw_v, vw_v, sem_k, sem_v, sem_l):
    wid = lax.axis_index("s") * 2 + lax.axis_index("c")
    pltpu.sync_copy(bounds_hbm, bnd_v)
    lanes = lax.iota(jnp.int32, 16)

    def bnd_at(i):
        vec = bnd_v[pl.ds((i >> 4) << 4, 16)]
        sel = jnp.where(lanes == (i & 15), vec, jnp.int32(-2147483647))
        return jnp.max(sel)

    def seg_body(sp, _):
        g = wid * SEGS_PER_TILE + sp
        seg_lo = pl.multiple_of(g * SEG, SEG)
        lo = bnd_at(g)
        hi = bnd_at(g + 1)
        lo_al = lo & jnp.int32(-8)
        trips = (hi - lo_al + (K - 1)) >> KSH
        pltpu.sync_copy(zeros_hbm, seg_v)

        def win_body(w, _):
            off = pl.multiple_of(jnp.minimum(lo_al + w * K, N - K), 8)
            look = pl.multiple_of(jnp.minimum(off + K, N - 16), 8)
            ck = pltpu.async_copy(keys_hbm.at[pl.ds(off, K)],
                                  kw_v.at[pl.ds(0, K)], sem_k)
            cv = pltpu.async_copy(vals_hbm.at[pl.ds(off, K)],
                                  vw_v.at[pl.ds(0, K)], sem_v)
            cl = pltpu.async_copy(keys_hbm.at[pl.ds(look, 16)],
                                  kw_v.at[pl.ds(K, 16)], sem_l)
            ck.wait()
            cv.wait()
            cl.wait()
            tail = kw_v[pl.ds(K, 16)]
            is_end = (off + K) >= N
            kw_v[pl.ds(K, 16)] = jnp.where(is_end, jnp.int32(M), tail)

            def vec_body(j, _):
                cur = kw_v[pl.ds(j * 16, 16)]
                nxt = kw_v[pl.ds(j * 16 + 1, 16)]
                vals = vw_v[pl.ds(j * 16, 16)]
                is_last = cur != nxt
                in_seg = (cur >= seg_lo) & (cur < seg_lo + SEG)
                m = is_last & in_seg
                plsc.store_scatter(seg_v, [cur - seg_lo], vals, mask=m)
                return 0

            lax.fori_loop(0, K // 16, vec_body, 0)
            return 0

        lax.fori_loop(0, trips, win_body, 0)
        pltpu.sync_copy(seg_v, out_hbm.at[pl.ds(seg_lo, SEG)])
        return 0

    lax.fori_loop(0, SEGS_PER_TILE, seg_body, 0)


def kernel(inputs, pooling_argmax):
    v = inputs.reshape(N)
    a = pooling_argmax.reshape(N)
    a_s, v_s = lax.sort((a, v), dimension=0, is_stable=False, num_keys=1)
    seg_starts = jnp.arange(NSEG + 1, dtype=jnp.int32) * SEG
    bounds = jnp.searchsorted(a_s, seg_starts, side="left").astype(jnp.int32)
    bounds = jnp.concatenate(
        [bounds, jnp.full((NB - NSEG - 1,), N, jnp.int32)])
    assert bounds.shape == (NB,)
    zeros = jnp.zeros((SEG,), jnp.float32)
    out = _scatter_sorted(a_s, v_s, bounds, zeros)
    return out.reshape(B, OUT_H, OUT_W, C)
